# baseline (speedup 1.0000x reference)
import functools

import jax
import jax.numpy as jnp
from jax import lax
from jax.experimental import pallas as pl
from jax.experimental.pallas import tpu as pltpu
from jax.experimental.pallas import tpu_sc as plsc

EMBED_DIM = 32
BATCH = 16384
_ROWS_PER_128 = 128 // EMBED_DIM

_NUM_CORES = 2
_NUM_SUBCORES = 16
_NW = _NUM_CORES * _NUM_SUBCORES
_B_PER_W = BATCH // _NW
_GROUPS = _B_PER_W // 16
_OUT_ROWS_W = _B_PER_W // _ROWS_PER_128


@functools.partial(
    pl.kernel,
    mesh=plsc.VectorSubcoreMesh(core_axis_name="c", subcore_axis_name="s"),
    out_type=jax.ShapeDtypeStruct((BATCH // _ROWS_PER_128, 128), jnp.float32),
    scratch_types=[
        pltpu.VMEM((_B_PER_W,), jnp.int32),
        pltpu.VMEM((_B_PER_W,), jnp.int32),
        pltpu.VMEM((_B_PER_W, 128), jnp.float32),
        pltpu.VMEM((_OUT_ROWS_W, 128), jnp.float32),
        pltpu.SemaphoreType.DMA,
    ],
)
def _gather_kernel(idx_hbm, table_hbm, out_hbm, idx_v, idx4_v, rows_v, out_v, sem):
    wid = lax.axis_index("s") * _NUM_CORES + lax.axis_index("c")
    base = wid * _B_PER_W
    pltpu.sync_copy(idx_hbm.at[pl.ds(base, _B_PER_W)], idx_v)

    def shift_body(i, carry):
        sl = pl.ds(i * 16, 16)
        idx4_v[sl] = lax.shift_right_logical(idx_v[sl], 2)
        return carry

    lax.fori_loop(0, _GROUPS, shift_body, 0)
    pltpu.async_copy(table_hbm.at[idx4_v], rows_v, sem).wait()

    def grp_body(g, carry):
        idxv = idx_v[pl.ds(g * 16, 16)]
        subx = (idxv & (_ROWS_PER_128 - 1)) * EMBED_DIM
        for j in range(16):
            r = g * 16 + j
            s0 = lax.index_in_dim(subx, j, keepdims=False)
            dst_r = g * (16 // _ROWS_PER_128) + (j // _ROWS_PER_128)
            cb = (j % _ROWS_PER_128) * EMBED_DIM
            for h in range(EMBED_DIM // 16):
                out_v[dst_r, pl.ds(cb + h * 16, 16)] = rows_v[r, pl.ds(s0 + h * 16, 16)]
        return carry

    lax.fori_loop(0, _GROUPS, grp_body, 0)
    pltpu.sync_copy(out_v, out_hbm.at[pl.ds(wid * _OUT_ROWS_W, _OUT_ROWS_W)])


def kernel(platform_ids, table):
    table2 = table.reshape(table.shape[0] * EMBED_DIM // 128, 128)
    out2 = _gather_kernel(platform_ids.astype(jnp.int32), table2)
    return out2.reshape(BATCH, EMBED_DIM)


# speedup vs baseline: 2.2437x; 2.2437x over previous
import functools

import jax
import jax.numpy as jnp
from jax import lax
from jax.experimental import pallas as pl
from jax.experimental.pallas import tpu as pltpu
from jax.experimental.pallas import tpu_sc as plsc

EMBED_DIM = 32
BATCH = 16384
NUM_ROWS = 100000

_NUM_CORES = 2
_NUM_SUBCORES = 16
_IDX_CHUNK = 4096


@functools.partial(
    pl.kernel,
    mesh=plsc.VectorSubcoreMesh(core_axis_name="c", subcore_axis_name="s"),
    out_type=jax.ShapeDtypeStruct((EMBED_DIM, BATCH), jnp.float32),
    scratch_types=[
        pltpu.VMEM((NUM_ROWS,), jnp.float32),
        pltpu.VMEM((_IDX_CHUNK,), jnp.int32),
        pltpu.VMEM((BATCH,), jnp.float32),
        pltpu.SemaphoreType.DMA,
        pltpu.SemaphoreType.DMA,
    ],
    compiler_params=pltpu.CompilerParams(needs_layout_passes=False),
)
def _gather_kernel(idx_hbm, tab_hbm, out_hbm, col_v, idx_v, row_v, csem, isem):
    d = lax.axis_index("s") * _NUM_CORES + lax.axis_index("c")
    col_dma = pltpu.async_copy(tab_hbm.at[d, :], col_v, csem)
    col_dma.wait()

    def chunk_body(j, carry):
        pltpu.async_copy(idx_hbm.at[pl.ds(j * _IDX_CHUNK, _IDX_CHUNK)], idx_v, isem).wait()

        def vec_body(k, carry2):
            iv = idx_v[pl.ds(k * 16, 16)]
            mask = iv >= 0
            row_v[pl.ds(j * _IDX_CHUNK + k * 16, 16)] = plsc.load_gather(
                col_v, [iv], mask=mask)
            return carry2

        lax.fori_loop(0, _IDX_CHUNK // 16, vec_body, 0)
        return carry

    lax.fori_loop(0, BATCH // _IDX_CHUNK, chunk_body, 0)
    pltpu.sync_copy(row_v, out_hbm.at[d, :])


def kernel(platform_ids, table):
    out_t = _gather_kernel(platform_ids.astype(jnp.int32), table.T)
    return out_t.T


# speedup vs baseline: 2.9856x; 1.3307x over previous
import functools

import jax
import jax.numpy as jnp
from jax import lax
from jax.experimental import pallas as pl
from jax.experimental.pallas import tpu as pltpu
from jax.experimental.pallas import tpu_sc as plsc

EMBED_DIM = 32
BATCH = 16384
NUM_ROWS = 100000

_NUM_CORES = 2
_NUM_SUBCORES = 16
_CHUNK = 4096
_NCHUNK = BATCH // _CHUNK
_NBUF = 3


@functools.partial(
    pl.kernel,
    mesh=plsc.VectorSubcoreMesh(core_axis_name="c", subcore_axis_name="s"),
    out_type=jax.ShapeDtypeStruct((EMBED_DIM, BATCH), jnp.float32),
    scratch_types=[
        pltpu.VMEM((NUM_ROWS,), jnp.float32),
        pltpu.VMEM((_CHUNK,), jnp.int32),
        pltpu.VMEM((_CHUNK,), jnp.int32),
        pltpu.VMEM((_CHUNK,), jnp.int32),
        pltpu.VMEM((BATCH,), jnp.float32),
        pltpu.SemaphoreType.DMA,
        pltpu.SemaphoreType.DMA,
        pltpu.SemaphoreType.DMA,
        pltpu.SemaphoreType.DMA,
        pltpu.SemaphoreType.DMA,
    ],
    compiler_params=pltpu.CompilerParams(needs_layout_passes=False),
)
def _gather_kernel(idx_hbm, tab_hbm, out_hbm, col_v, idx_v0, idx_v1, idx_v2,
                   row_v, csem, isem0, isem1, isem2, wsem):
    d = lax.axis_index("s") * _NUM_CORES + lax.axis_index("c")
    col_cp = pltpu.async_copy(tab_hbm.at[d, :], col_v, csem)

    ibufs = [idx_v0, idx_v1, idx_v2]
    isems = [isem0, isem1, isem2]

    def fetch(c):
        return pltpu.async_copy(
            idx_hbm.at[pl.ds(c * _CHUNK, _CHUNK)], ibufs[c % _NBUF], isems[c % _NBUF])

    pending = {c: fetch(c) for c in range(min(_NBUF, _NCHUNK))}
    col_cp.wait()

    write_cps = []
    for c in range(_NCHUNK):
        pending[c].wait()
        buf = ibufs[c % _NBUF]

        @plsc.parallel_loop(0, _CHUNK, step=16, unroll=8)
        def gather_body(i):
            iv = buf[pl.ds(i, 16)]
            row_v[pl.ds(c * _CHUNK + i, 16)] = plsc.load_gather(col_v, [iv])

        if c + _NBUF < _NCHUNK:
            pending[c + _NBUF] = fetch(c + _NBUF)
        write_cps.append(pltpu.async_copy(
            row_v.at[pl.ds(c * _CHUNK, _CHUNK)],
            out_hbm.at[d, pl.ds(c * _CHUNK, _CHUNK)], wsem))
    for cp in write_cps:
        cp.wait()


def kernel(platform_ids, table):
    out_t = _gather_kernel(platform_ids.astype(jnp.int32), table.T)
    return out_t.T
